# baseline (speedup 1.0000x reference)
import functools

import jax
import jax.numpy as jnp
from jax.experimental import pallas as pl
from jax.experimental.pallas import tpu as pltpu


def _cbow_body(ids_ref, emb_ref, w_ref, b_ref, out_ref,
               wbf_ref, emb4_ref, gt_ref, ctx_ref, *, C, TB, TV, P, S, V):
    i = pl.program_id(1)
    j = pl.program_id(2)

    @pl.when(jnp.logical_and(i == 0, j == 0))
    def _retile():
        def copy64(t, carry):
            base = pl.multiple_of(t * 64, 64)
            for k in range(P):
                emb4_ref[pl.Slice(base * P + k, 64, P), :] = (
                    emb_ref[pl.ds(base, 64), 128 * k:128 * (k + 1)])
            return carry
        jax.lax.fori_loop(0, V // 64, copy64, 0, unroll=False)

    @pl.when(i == 0)
    def _cast_w():
        sl = pl.ds(pl.multiple_of(j * TV, 8), TV)
        wbf_ref[sl, :] = w_ref[sl, :].astype(jnp.bfloat16)

    @pl.when(j == 0)
    def _gather():
        def group8(g, carry):
            for r in range(8):
                base = (g * 8 + r) * C
                idx = pl.multiple_of(ids_ref[base], P)
                acc = emb4_ref[pl.ds(idx, P), :]
                for c in range(1, C):
                    idx = pl.multiple_of(ids_ref[base + c], P)
                    acc = acc + emb4_ref[pl.ds(idx, P), :]
                gt_ref[pl.Slice(g * 8 + r, P, S), :] = acc
            return carry

        jax.lax.fori_loop(0, TB // 8, group8, 0, unroll=False)
        for k in range(P):
            ctx_ref[:, k * 128:(k + 1) * 128] = (
                gt_ref[pl.ds(k * S, TB), :].astype(jnp.bfloat16))

    wt = wbf_ref[pl.ds(pl.multiple_of(j * TV, 8), TV), :]
    logits = jax.lax.dot_general(
        ctx_ref[...], wt,
        dimension_numbers=(((1,), (1,)), ((), ())),
        preferred_element_type=jnp.float32)
    out_ref[...] = logits + b_ref[...]


def kernel(context_words, emb_table, linear_w, linear_b):
    B, C = context_words.shape
    V, D = emb_table.shape
    assert linear_w.shape == (V, D) and linear_b.shape == (V,)
    assert V % 128 == 0 and D % 128 == 0

    P = D // 128
    NC = 2
    TB = min(256, B // NC)
    TV = min(1024, V)
    nb = B // (TB * NC)
    assert B % (TB * NC) == 0 and V % TV == 0 and TB % 8 == 0 and V % 64 == 0
    S = TB + 1

    ids = jnp.clip(context_words.reshape(-1).astype(jnp.int32), 0, V - 1) * P
    b2d = linear_b.reshape(1, V)

    body = functools.partial(_cbow_body, C=C, TB=TB, TV=TV, P=P, S=S, V=V)
    return pl.pallas_call(
        body,
        out_shape=jax.ShapeDtypeStruct((B, V), jnp.float32),
        grid=(NC, nb, V // TV),
        in_specs=[
            pl.BlockSpec((TB * C,), lambda c, i, j, nb=nb: (c * nb + i,),
                         memory_space=pltpu.MemorySpace.SMEM),
            pl.BlockSpec((V, D), lambda c, i, j: (0, 0),
                         pipeline_mode=pl.Buffered(1)),
            pl.BlockSpec((V, D), lambda c, i, j: (0, 0),
                         pipeline_mode=pl.Buffered(1)),
            pl.BlockSpec((1, TV), lambda c, i, j: (0, j)),
        ],
        out_specs=pl.BlockSpec((TB, TV), lambda c, i, j, nb=nb: (c * nb + i, j)),
        scratch_shapes=[
            pltpu.VMEM((V, D), jnp.bfloat16),
            pltpu.VMEM((V * P, 128), jnp.float32),
            pltpu.VMEM((S * P, 128), jnp.float32),
            pltpu.VMEM((TB, D), jnp.bfloat16),
        ],
        compiler_params=pltpu.CompilerParams(
            dimension_semantics=("parallel", "arbitrary", "arbitrary"),
            vmem_limit_bytes=64 << 20),
    )(ids, emb_table, linear_w, b2d)


# speedup vs baseline: 1.2583x; 1.2583x over previous
import functools

import jax
import jax.numpy as jnp
from jax.experimental import pallas as pl
from jax.experimental.pallas import tpu as pltpu


def _cbow_body(ids_ref, emb_ref, w_ref, b_ref, out_ref,
               wbf_ref, emb4_ref, gt_ref, ctx_ref, *, C, TB, TV, P, S, V):
    i = pl.program_id(1)
    j = pl.program_id(2)

    @pl.when(jnp.logical_and(i == 0, j == 0))
    def _retile():
        def copy64(t, carry):
            base = pl.multiple_of(t * 64, 64)
            for k in range(P):
                emb4_ref[pl.Slice(base * P + k, 64, P), :] = (
                    emb_ref[pl.ds(base, 64), 128 * k:128 * (k + 1)])
            return carry
        jax.lax.fori_loop(0, V // 64, copy64, 0, unroll=False)

    @pl.when(i == 0)
    def _cast_w():
        sl = pl.ds(pl.multiple_of(j * TV, 8), TV)
        wbf_ref[sl, :] = w_ref[sl, :].astype(jnp.bfloat16)

    @pl.when(j == 0)
    def _gather():
        def group8(g, carry):
            for r in range(8):
                base = (g * 8 + r) * C
                idx = pl.multiple_of(ids_ref[base], P)
                acc = emb4_ref[pl.ds(idx, P), :]
                for c in range(1, C):
                    idx = pl.multiple_of(ids_ref[base + c], P)
                    acc = acc + emb4_ref[pl.ds(idx, P), :]
                gt_ref[pl.Slice(g * 8 + r, P, S), :] = acc
            return carry

        jax.lax.fori_loop(0, TB // 8, group8, 0, unroll=False)
        for k in range(P):
            ctx_ref[:, k * 128:(k + 1) * 128] = (
                gt_ref[pl.ds(k * S, TB), :].astype(jnp.bfloat16))

    wt = wbf_ref[pl.ds(pl.multiple_of(j * TV, 8), TV), :]
    logits = jax.lax.dot_general(
        ctx_ref[...], wt,
        dimension_numbers=(((1,), (1,)), ((), ())),
        preferred_element_type=jnp.float32)
    out_ref[...] = logits + b_ref[...]


def kernel(context_words, emb_table, linear_w, linear_b):
    B, C = context_words.shape
    V, D = emb_table.shape
    assert linear_w.shape == (V, D) and linear_b.shape == (V,)
    assert V % 128 == 0 and D % 128 == 0

    P = D // 128
    NC = 2
    TB = min(256, B // NC)
    TV = min(2048, V)
    nb = B // (TB * NC)
    assert B % (TB * NC) == 0 and V % TV == 0 and TB % 8 == 0 and V % 64 == 0
    S = TB + 1

    ids = jnp.clip(context_words.reshape(-1).astype(jnp.int32), 0, V - 1) * P
    b2d = linear_b.reshape(1, V)

    body = functools.partial(_cbow_body, C=C, TB=TB, TV=TV, P=P, S=S, V=V)
    return pl.pallas_call(
        body,
        out_shape=jax.ShapeDtypeStruct((B, V), jnp.float32),
        grid=(NC, nb, V // TV),
        in_specs=[
            pl.BlockSpec((TB * C,), lambda c, i, j, nb=nb: (c * nb + i,),
                         memory_space=pltpu.MemorySpace.SMEM),
            pl.BlockSpec((V, D), lambda c, i, j: (0, 0),
                         pipeline_mode=pl.Buffered(1)),
            pl.BlockSpec((V, D), lambda c, i, j: (0, 0),
                         pipeline_mode=pl.Buffered(1)),
            pl.BlockSpec((1, TV), lambda c, i, j: (0, j)),
        ],
        out_specs=pl.BlockSpec((TB, TV), lambda c, i, j, nb=nb: (c * nb + i, j)),
        scratch_shapes=[
            pltpu.VMEM((V, D), jnp.bfloat16),
            pltpu.VMEM((V * P, 128), jnp.float32),
            pltpu.VMEM((S * P, 128), jnp.float32),
            pltpu.VMEM((TB, D), jnp.bfloat16),
        ],
        compiler_params=pltpu.CompilerParams(
            dimension_semantics=("parallel", "arbitrary", "arbitrary"),
            vmem_limit_bytes=64 << 20),
    )(ids, emb_table, linear_w, b2d)


# speedup vs baseline: 1.2787x; 1.0162x over previous
import functools

import jax
import jax.numpy as jnp
from jax.experimental import pallas as pl
from jax.experimental.pallas import tpu as pltpu


def _cbow_body(ids_ref, idsn_ref, emb_ref, w_ref, b_ref, out_ref,
               wbf_ref, emb4_ref, gt_ref, ctx_ref, *, C, TB, TV, P, S, V, NB, NV):
    i = pl.program_id(1)
    j = pl.program_id(2)

    @pl.when(jnp.logical_and(i == 0, j == 0))
    def _retile():
        def copy64(t, carry):
            base = pl.multiple_of(t * 64, 64)
            for k in range(P):
                emb4_ref[pl.Slice(base * P + k, 64, P), :] = (
                    emb_ref[pl.ds(base, 64), 128 * k:128 * (k + 1)])
            return carry
        jax.lax.fori_loop(0, V // 64, copy64, 0, unroll=False)

    @pl.when(i == 0)
    def _cast_w():
        sl = pl.ds(pl.multiple_of(j * TV, 8), TV)
        wbf_ref[sl, :] = w_ref[sl, :].astype(jnp.bfloat16)

    def _gather_span(ids, row0, n):
        def group8(g, carry):
            for r in range(8):
                row = row0 + g * 8 + r
                base = row * C
                idx = pl.multiple_of(ids[base], P)
                acc = emb4_ref[pl.ds(idx, P), :]
                for c in range(1, C):
                    idx = pl.multiple_of(ids[base + c], P)
                    acc = acc + emb4_ref[pl.ds(idx, P), :]
                gt_ref[pl.Slice(row, P, S), :] = acc
            return carry

        jax.lax.fori_loop(0, n // 8, group8, 0, unroll=False)

    @pl.when(jnp.logical_and(i == 0, j == 0))
    def _gather_first():
        _gather_span(ids_ref, 0, TB)

    @pl.when(j == 0)
    def _assemble():
        for k in range(P):
            ctx_ref[:, k * 128:(k + 1) * 128] = (
                gt_ref[pl.ds(k * S, TB), :].astype(jnp.bfloat16))

    @pl.when(i + 1 < NB)
    def _gather_next():
        _gather_span(idsn_ref, j * (TB // NV), TB // NV)

    wt = wbf_ref[pl.ds(pl.multiple_of(j * TV, 8), TV), :]
    logits = jax.lax.dot_general(
        ctx_ref[...], wt,
        dimension_numbers=(((1,), (1,)), ((), ())),
        preferred_element_type=jnp.float32)
    out_ref[...] = logits + b_ref[...]


def kernel(context_words, emb_table, linear_w, linear_b):
    B, C = context_words.shape
    V, D = emb_table.shape
    assert linear_w.shape == (V, D) and linear_b.shape == (V,)
    assert V % 128 == 0 and D % 128 == 0

    P = D // 128
    NC = 2
    TB = min(256, B // NC)
    TV = min(2048, V)
    nb = B // (TB * NC)
    assert B % (TB * NC) == 0 and V % TV == 0 and TB % 8 == 0 and V % 64 == 0
    assert (TB // (V // TV)) % 8 == 0
    S = TB + 1

    ids = jnp.clip(context_words.reshape(-1).astype(jnp.int32), 0, V - 1) * P
    b2d = linear_b.reshape(1, V)

    body = functools.partial(_cbow_body, C=C, TB=TB, TV=TV, P=P, S=S, V=V,
                             NB=nb, NV=V // TV)
    return pl.pallas_call(
        body,
        out_shape=jax.ShapeDtypeStruct((B, V), jnp.float32),
        grid=(NC, nb, V // TV),
        in_specs=[
            pl.BlockSpec((TB * C,), lambda c, i, j, nb=nb: (c * nb + i,),
                         memory_space=pltpu.MemorySpace.SMEM),
            pl.BlockSpec((TB * C,),
                         lambda c, i, j, nb=nb: (
                             c * nb + jnp.minimum(i + 1, nb - 1),),
                         memory_space=pltpu.MemorySpace.SMEM),
            pl.BlockSpec((V, D), lambda c, i, j: (0, 0),
                         pipeline_mode=pl.Buffered(1)),
            pl.BlockSpec((V, D), lambda c, i, j: (0, 0),
                         pipeline_mode=pl.Buffered(1)),
            pl.BlockSpec((1, TV), lambda c, i, j: (0, j)),
        ],
        out_specs=pl.BlockSpec((TB, TV), lambda c, i, j, nb=nb: (c * nb + i, j)),
        scratch_shapes=[
            pltpu.VMEM((V, D), jnp.bfloat16),
            pltpu.VMEM((V * P, 128), jnp.float32),
            pltpu.VMEM((S * P, 128), jnp.float32),
            pltpu.VMEM((TB, D), jnp.bfloat16),
        ],
        compiler_params=pltpu.CompilerParams(
            dimension_semantics=("parallel", "arbitrary", "arbitrary"),
            vmem_limit_bytes=64 << 20),
    )(ids, ids, emb_table, linear_w, b2d)
